# Initial kernel scaffold; baseline (speedup 1.0000x reference)
import jax
import jax.numpy as jnp
from jax.experimental import pallas as pl


def kernel(x, Wg, W1, b1, W2, b2):
    raise NotImplementedError("write your pallas kernel here")


# speedup vs baseline: 1.8533x; 1.8533x over previous
import functools

import jax
import jax.numpy as jnp
from jax.experimental import pallas as pl

D_MODEL = 1024
D_EXPERT = 512
NUM_EXPERTS = 8
BT = 512


def _moe_block(x_ref, wg_ref, w1_ref, b1_ref, w2_ref, b2_ref, out_ref):
    xb = x_ref[...]
    logits = jax.lax.dot_general(
        xb, wg_ref[...], (((1,), (1,)), ((), ())),
        preferred_element_type=jnp.float32)
    m = jnp.max(logits, axis=-1, keepdims=True)
    ex = jnp.exp(logits - m)
    probs = ex / jnp.sum(ex, axis=-1, keepdims=True)

    e0 = jnp.argmax(probs, axis=-1)
    w0 = jnp.max(probs, axis=-1)
    iota = jax.lax.broadcasted_iota(jnp.int32, probs.shape, 1)
    probs2 = jnp.where(iota == e0[:, None], -jnp.inf, probs)
    e1 = jnp.argmax(probs2, axis=-1)
    w1 = jnp.max(probs2, axis=-1)
    s = w0 + w1
    w0 = w0 / s
    w1 = w1 / s

    acc = jnp.zeros((xb.shape[0], D_MODEL), dtype=jnp.float32)
    for e in range(NUM_EXPERTS):
        h = jax.lax.dot_general(
            xb, w1_ref[e], (((1,), (1,)), ((), ())),
            preferred_element_type=jnp.float32) + b1_ref[e][None, :]
        h = jnp.maximum(h, 0.0)
        y = jax.lax.dot_general(
            h, w2_ref[e], (((1,), (1,)), ((), ())),
            preferred_element_type=jnp.float32) + b2_ref[e][None, :]
        cw = jnp.where(e0 == e, w0, 0.0) + jnp.where(e1 == e, w1, 0.0)
        acc = acc + y * cw[:, None]
    out_ref[...] = acc


@jax.jit
def kernel(x, Wg, W1, b1, W2, b2):
    B, S, D = x.shape
    T = B * S
    xf = x.reshape(T, D)
    grid = (T // BT,)
    out = pl.pallas_call(
        _moe_block,
        grid=grid,
        in_specs=[
            pl.BlockSpec((BT, D), lambda i: (i, 0)),
            pl.BlockSpec((NUM_EXPERTS, D), lambda i: (0, 0)),
            pl.BlockSpec((NUM_EXPERTS, D_EXPERT, D), lambda i: (0, 0, 0)),
            pl.BlockSpec((NUM_EXPERTS, D_EXPERT), lambda i: (0, 0)),
            pl.BlockSpec((NUM_EXPERTS, D, D_EXPERT), lambda i: (0, 0, 0)),
            pl.BlockSpec((NUM_EXPERTS, D), lambda i: (0, 0)),
        ],
        out_specs=pl.BlockSpec((BT, D), lambda i: (i, 0)),
        out_shape=jax.ShapeDtypeStruct((T, D), jnp.float32),
    )(xf, Wg, W1, b1, W2, b2)
    return out.reshape(B, S, D)
